# baseline (speedup 1.0000x reference)
import functools

import numpy as np
import jax
import jax.numpy as jnp
from jax import lax
from jax.experimental import pallas as pl
from jax.experimental.pallas import tpu as pltpu
from jax.experimental.pallas import tpu_sc as plsc

_BATCH = 4096
_NUM_GROUP = 2
_TAU = 0.2
_SAMP = _BATCH // _NUM_GROUP

_NC = 2
_NS = 16
_NW = _NC * _NS
_BPW = _BATCH // _NW
_SCH = 16
_GCH = 8


def _rng_constants():
    key = jax.random.key(42)
    key, s0 = jax.random.split(key)
    key, s1 = jax.random.split(key)
    key, s2 = jax.random.split(key)
    u0 = np.asarray(jax.random.uniform(s0, (_SAMP,), jnp.float32))
    u1 = np.asarray(jax.random.uniform(s1, (_SAMP,), jnp.float32))
    perm = np.asarray(jax.random.permutation(s2, _BATCH))
    gf = (perm >= _SAMP).astype(np.int32)
    up = np.where(gf, u1[np.minimum(perm - _SAMP, _SAMP - 1)],
                  u0[np.minimum(perm, _SAMP - 1)]).astype(np.float32)
    return up, gf


_UP, _GF = _rng_constants()

_mesh = plsc.VectorSubcoreMesh(core_axis_name="c", subcore_axis_name="s")

_SEARCH_STEPS = [2048, 1024, 512, 256, 128, 64, 32, 16, 8, 4, 2, 1]


@functools.partial(
    pl.kernel,
    mesh=_mesh,
    out_type=(
        jax.ShapeDtypeStruct((_BATCH, 2048), jnp.float32),
        jax.ShapeDtypeStruct((_BATCH, 1024), jnp.float32),
        jax.ShapeDtypeStruct((_BATCH, 768), jnp.float32),
        jax.ShapeDtypeStruct((_BATCH, 128), jnp.int32),
    ),
    scratch_types=[
        pltpu.VMEM((2 * _BATCH + 16,), jnp.float32),
        pltpu.VMEM((2 * _BATCH + 16,), jnp.int32),
        pltpu.VMEM((_BPW,), jnp.float32),
        pltpu.VMEM((_BPW,), jnp.int32),
        pltpu.VMEM((_BPW,), jnp.int32),
        pltpu.VMEM((_BPW, 128), jnp.int32),
        pltpu.SemaphoreType.DMA,
        pltpu.SemaphoreType.DMA,
        pltpu.SemaphoreType.DMA,
        pltpu.SemaphoreType.DMA,
    ],
)
def _sc_sample_gather(video_hbm, audio_hbm, text_hbm, tg_hbm,
                      p0_hbm, p1_hbm, idx0_hbm, idx1_hbm, up_hbm, gf_hbm,
                      out_v, out_a, out_t, out_tg,
                      ptab, itab, ubuf, gbuf, ibuf, tg_buf,
                      sem_v, sem_a, sem_t, sem_tg):
    wid = lax.axis_index("s") * _NC + lax.axis_index("c")
    base = wid * _BPW

    pltpu.sync_copy(p0_hbm, ptab.at[pl.ds(0, _BATCH)])
    pltpu.sync_copy(p1_hbm, ptab.at[pl.ds(_BATCH, _BATCH)])
    pltpu.sync_copy(idx0_hbm, itab.at[pl.ds(0, _BATCH)])
    pltpu.sync_copy(idx1_hbm, itab.at[pl.ds(_BATCH, _BATCH)])
    pltpu.sync_copy(up_hbm.at[pl.ds(base, _BPW)], ubuf)
    pltpu.sync_copy(gf_hbm.at[pl.ds(base, _BPW)], gbuf)

    t0 = ptab[pl.ds(_BATCH - 16, 16)][15]
    t1 = ptab[pl.ds(2 * _BATCH - 16, 16)][15]
    iota = lax.broadcasted_iota(jnp.int32, (16,), 0)

    def search_body(c, carry):
        co = pl.multiple_of(c * _SCH, 16)
        upv = ubuf[pl.ds(co, 16)]
        gv = gbuf[pl.ds(co, 16)]
        rv = jnp.where(gv == 1, t1, t0) * (jnp.float32(1.0) - upv)
        tb = [gv[l] * _BATCH for l in range(16)]
        rs = [rv[l] for l in range(16)]
        pos = [jnp.int32(0)] * 16
        for k in _SEARCH_STEPS:
            for l in range(16):
                probe = ptab[pl.ds(tb[l] + pos[l] + (k - 1), 16)][0]
                pos[l] = jnp.where(probe < rs[l], pos[l] + k, pos[l])
        acc = jnp.zeros((16,), jnp.int32)
        members = []
        for l in range(16):
            member = itab[pl.ds(tb[l] + pos[l], 16)][0]
            members.append(member)
            acc = acc + jnp.where(iota == l, member, 0)
        ibuf[pl.ds(co, 16)] = acc
        for l in range(16):
            m = members[l]
            row = base + co + l
            pltpu.async_copy(video_hbm.at[pl.ds(m, 1)],
                             out_v.at[pl.ds(row, 1)], sem_v)
            pltpu.async_copy(audio_hbm.at[pl.ds(m, 1)],
                             out_a.at[pl.ds(row, 1)], sem_a)
            pltpu.async_copy(text_hbm.at[pl.ds(m, 1)],
                             out_t.at[pl.ds(row, 1)], sem_t)
        return carry

    lax.fori_loop(0, _BPW // _SCH, search_body, jnp.int32(0))

    ctg = pltpu.async_copy(tg_hbm.at[ibuf], tg_buf, sem_tg)
    ctg.wait()
    pltpu.sync_copy(tg_buf, out_tg.at[pl.ds(base, _BPW)])

    pltpu.make_async_copy(video_hbm.at[pl.ds(0, _BPW)],
                          out_v.at[pl.ds(base, _BPW)], sem_v).wait()
    pltpu.make_async_copy(audio_hbm.at[pl.ds(0, _BPW)],
                          out_a.at[pl.ds(base, _BPW)], sem_a).wait()
    pltpu.make_async_copy(text_hbm.at[pl.ds(0, _BPW)],
                          out_t.at[pl.ds(base, _BPW)], sem_t).wait()


def kernel(batch_video, batch_audio, batch_text, batch_target, batch_group,
           batch_group_others):
    positions = jnp.arange(_BATCH)
    p_cumls = []
    idx_gs = []
    for i in range(_NUM_GROUP):
        mask = batch_group == i
        idx_g = jnp.nonzero(mask, size=_BATCH, fill_value=0)[0]
        n_c = jnp.sum(mask)
        n_c_f = n_c.astype(jnp.float32)
        weights_list = []
        for j in range(4):
            n = jnp.sum(mask & (batch_group_others == j)).astype(jnp.float32)
            weights_list.append((n / n_c_f) ** _TAU)
        tot = sum(weights_list)
        weights_list = [w / tot for w in weights_list]
        group_others = batch_group_others[idx_g]
        w_arr = jnp.stack(weights_list).astype(jnp.float32)[group_others]
        w_arr = jnp.where(positions < n_c, w_arr, 0.0)
        probs = w_arr / w_arr.sum()
        p_cumls.append(jnp.cumsum(probs))
        idx_gs.append(idx_g.astype(jnp.int32))

    tg_packed = jnp.pad(
        jnp.stack([batch_target, batch_group], axis=-1), ((0, 0), (0, 126)))
    video, audio, text, tg_out = _sc_sample_gather(
        batch_video, batch_audio, batch_text, tg_packed,
        p_cumls[0], p_cumls[1], idx_gs[0], idx_gs[1],
        jnp.asarray(_UP), jnp.asarray(_GF))
    return (video, audio, text, tg_out[:, 0], tg_out[:, 1])


# speedup vs baseline: 17.6982x; 17.6982x over previous
import functools

import numpy as np
import jax
import jax.numpy as jnp
from jax import lax
from jax.experimental import pallas as pl
from jax.experimental.pallas import tpu as pltpu
from jax.experimental.pallas import tpu_sc as plsc

_BATCH = 4096
_NUM_GROUP = 2
_TAU = 0.2
_SAMP = _BATCH // _NUM_GROUP

_NC = 2
_NS = 16
_NW = _NC * _NS
_BPW = _BATCH // _NW
_SCH = 16
_GCH = 8


def _rng_constants():
    key = jax.random.key(42)
    key, s0 = jax.random.split(key)
    key, s1 = jax.random.split(key)
    key, s2 = jax.random.split(key)
    u0 = np.asarray(jax.random.uniform(s0, (_SAMP,), jnp.float32))
    u1 = np.asarray(jax.random.uniform(s1, (_SAMP,), jnp.float32))
    perm = np.asarray(jax.random.permutation(s2, _BATCH))
    gf = (perm >= _SAMP).astype(np.int32)
    up = np.where(gf, u1[np.minimum(perm - _SAMP, _SAMP - 1)],
                  u0[np.minimum(perm, _SAMP - 1)]).astype(np.float32)
    return up, gf


_UP, _GF = _rng_constants()

_mesh = plsc.VectorSubcoreMesh(core_axis_name="c", subcore_axis_name="s")

_SEARCH_STEPS = [2048, 1024, 512, 256, 128, 64, 32, 16, 8, 4, 2, 1]


@functools.partial(
    pl.kernel,
    mesh=_mesh,
    out_type=(
        jax.ShapeDtypeStruct((_BATCH,), jnp.int32),
        jax.ShapeDtypeStruct((_BATCH,), jnp.int32),
        jax.ShapeDtypeStruct((_BATCH,), jnp.float32),
        jax.ShapeDtypeStruct((_BATCH,), jnp.float32),
    ),
    scratch_types=[
        pltpu.VMEM((_BATCH + 16,), jnp.int32),
        pltpu.VMEM((_BATCH + 16,), jnp.int32),
        pltpu.VMEM((_BATCH + 16,), jnp.float32),
        pltpu.VMEM((_BATCH + 16,), jnp.float32),
        pltpu.VMEM((256,), jnp.int32),
        pltpu.VMEM((256,), jnp.int32),
        pltpu.VMEM((256,), jnp.float32),
        pltpu.VMEM((256,), jnp.int32),
        pltpu.VMEM((256,), jnp.float32),
        pltpu.VMEM((256,), jnp.int32),
        pltpu.VMEM((256,), jnp.float32),
        pltpu.VMEM((16,), jnp.int32),
        pltpu.VMEM((_BPW,), jnp.int32),
        pltpu.VMEM((_BPW,), jnp.float32),
        pltpu.VMEM((_BPW,), jnp.int32),
        pltpu.VMEM((_BPW,), jnp.float32),
        pltpu.SMEM((257,), jnp.int32),
        pltpu.SMEM((257,), jnp.int32),
        pltpu.SMEM((257,), jnp.float32),
        pltpu.SMEM((257,), jnp.float32),
        pltpu.SMEM((_NS + 1,), jnp.int32),
        pltpu.VMEM_SHARED((_BATCH,), jnp.int32),
        pltpu.VMEM_SHARED((_BATCH,), jnp.int32),
        pltpu.VMEM_SHARED((_BATCH,), jnp.float32),
        pltpu.VMEM_SHARED((_BATCH,), jnp.float32),
        pltpu.VMEM_SHARED((256,), jnp.int32),
    ],
)
def _sc_compact(group_hbm, wfull_hbm,
                idx0_out, idx1_out, w0_out, w1_out,
                ri0, ri1, rw0, rw1, cnttab, gloc, wloc, st_i0, st_w0,
                st_i1, st_w1, cvreg, mi0, mw0, mi1, mw1,
                si0, si1, sw0, sw1, spref,
                sh_i0, sh_i1, sh_w0, sh_w1, sh_cnt):
    scid = lax.axis_index("c")
    sid = lax.axis_index("s")
    ibase = pl.multiple_of(sid * 256, 256)
    obase = pl.multiple_of(scid * (_BATCH // 2) + sid * _BPW, _BPW)

    pltpu.sync_copy(group_hbm.at[pl.ds(ibase, 256)], gloc)
    pltpu.sync_copy(wfull_hbm.at[pl.ds(ibase, 256)], wloc)

    iota = lax.broadcasted_iota(jnp.int32, (16,), 0)

    def compact_body(j, carry):
        c0, c1 = carry
        jo = pl.multiple_of(j * 16, 16)
        gv = gloc[pl.ds(jo, 16)]
        wv = wloc[pl.ds(jo, 16)]
        for l in range(16):
            pos = ibase + jo + l
            w_s = wv[l]
            e0 = gv[l] == 0
            si0[c0] = jnp.where(e0, pos, -1)
            sw0[c0] = jnp.where(e0, w_s, 0.0)
            si1[c1] = jnp.where(e0, -1, pos)
            sw1[c1] = jnp.where(e0, 0.0, w_s)
            c0 = jnp.where(e0, c0 + 1, c0)
            c1 = jnp.where(e0, c1, c1 + 1)
        return (c0, c1)

    c0, c1 = lax.fori_loop(0, 16, compact_body,
                           (jnp.int32(0), jnp.int32(0)))

    def asm_body(j, carry):
        jo = pl.multiple_of(j * 16, 16)
        a_i0 = jnp.zeros((16,), jnp.int32)
        a_i1 = jnp.zeros((16,), jnp.int32)
        a_w0 = jnp.zeros((16,), jnp.float32)
        a_w1 = jnp.zeros((16,), jnp.float32)
        for l in range(16):
            a_i0 = a_i0 + jnp.where(iota == l, si0[jo + l], 0)
            a_i1 = a_i1 + jnp.where(iota == l, si1[jo + l], 0)
            a_w0 = a_w0 + jnp.where(iota == l, sw0[jo + l], 0.0)
            a_w1 = a_w1 + jnp.where(iota == l, sw1[jo + l], 0.0)
        st_i0[pl.ds(jo, 16)] = a_i0
        st_w0[pl.ds(jo, 16)] = a_w0
        st_i1[pl.ds(jo, 16)] = a_i1
        st_w1[pl.ds(jo, 16)] = a_w1
        return carry

    lax.fori_loop(0, 16, asm_body, jnp.int32(0))
    pltpu.sync_copy(st_i0, sh_i0.at[pl.ds(ibase, 256)])
    pltpu.sync_copy(st_w0, sh_w0.at[pl.ds(ibase, 256)])
    pltpu.sync_copy(st_i1, sh_i1.at[pl.ds(ibase, 256)])
    pltpu.sync_copy(st_w1, sh_w1.at[pl.ds(ibase, 256)])
    cvreg[...] = jnp.where(iota == 0, c0, 0)
    pltpu.sync_copy(cvreg, sh_cnt.at[pl.ds(pl.multiple_of(sid * 16, 16), 16)])
    plsc.subcore_barrier()

    pltpu.sync_copy(sh_i0, ri0.at[pl.ds(0, _BATCH)])
    pltpu.sync_copy(sh_i1, ri1.at[pl.ds(0, _BATCH)])
    pltpu.sync_copy(sh_w0, rw0.at[pl.ds(0, _BATCH)])
    pltpu.sync_copy(sh_w1, rw1.at[pl.ds(0, _BATCH)])
    pltpu.sync_copy(sh_cnt, cnttab)
    pref = jnp.int32(0)
    for w in range(_NS):
        spref[w] = pref
        pref = pref + cnttab[pl.ds(w * 16, 16)][0]
    spref[_NS] = pref

    def merge_body(j, carry):
        co = pl.multiple_of(j * 16, 16)
        a_i0 = jnp.zeros((16,), jnp.int32)
        a_i1 = jnp.zeros((16,), jnp.int32)
        a_w0 = jnp.zeros((16,), jnp.float32)
        a_w1 = jnp.zeros((16,), jnp.float32)
        for l in range(16):
            k = obase + co + l
            w0p = jnp.int32(0)
            w1p = jnp.int32(0)
            for st in (8, 4, 2, 1):
                c0n = w0p + st
                w0p = jnp.where((c0n <= _NS - 1) & (spref[c0n] <= k), c0n, w0p)
                c1n = w1p + st
                w1p = jnp.where(
                    (c1n <= _NS - 1) & (c1n * 256 - spref[c1n] <= k), c1n, w1p)
            off0 = jnp.minimum(k - spref[w0p], 255)
            src0 = w0p * 256 + off0
            a_i0 = a_i0 + jnp.where(iota == l, ri0[pl.ds(src0, 16)][0], 0)
            a_w0 = a_w0 + jnp.where(iota == l, rw0[pl.ds(src0, 16)][0], 0.0)
            off1 = jnp.minimum(k - (w1p * 256 - spref[w1p]), 255)
            src1 = w1p * 256 + off1
            a_i1 = a_i1 + jnp.where(iota == l, ri1[pl.ds(src1, 16)][0], 0)
            a_w1 = a_w1 + jnp.where(iota == l, rw1[pl.ds(src1, 16)][0], 0.0)
        mi0[pl.ds(co, 16)] = a_i0
        mw0[pl.ds(co, 16)] = a_w0
        mi1[pl.ds(co, 16)] = a_i1
        mw1[pl.ds(co, 16)] = a_w1
        return carry

    lax.fori_loop(0, _BPW // 16, merge_body, jnp.int32(0))
    pltpu.sync_copy(mi0, idx0_out.at[pl.ds(obase, _BPW)])
    pltpu.sync_copy(mw0, w0_out.at[pl.ds(obase, _BPW)])
    pltpu.sync_copy(mi1, idx1_out.at[pl.ds(obase, _BPW)])
    pltpu.sync_copy(mw1, w1_out.at[pl.ds(obase, _BPW)])


@functools.partial(
    pl.kernel,
    mesh=_mesh,
    out_type=(
        jax.ShapeDtypeStruct((_BATCH, 2048), jnp.float32),
        jax.ShapeDtypeStruct((_BATCH, 1024), jnp.float32),
        jax.ShapeDtypeStruct((_BATCH, 768), jnp.float32),
        jax.ShapeDtypeStruct((_BATCH, 128), jnp.int32),
    ),
    scratch_types=[
        pltpu.VMEM((2 * _BATCH + 16,), jnp.float32),
        pltpu.VMEM((2 * _BATCH + 16,), jnp.int32),
        pltpu.VMEM((_BPW,), jnp.float32),
        pltpu.VMEM((_BPW,), jnp.int32),
        pltpu.VMEM((_BPW,), jnp.int32),
        pltpu.VMEM((_GCH, 2048), jnp.float32),
        pltpu.VMEM((_GCH, 2048), jnp.float32),
        pltpu.VMEM((_GCH, 1024), jnp.float32),
        pltpu.VMEM((_GCH, 1024), jnp.float32),
        pltpu.VMEM((_GCH, 768), jnp.float32),
        pltpu.VMEM((_GCH, 768), jnp.float32),
        pltpu.VMEM((_GCH, 128), jnp.int32),
        pltpu.VMEM((_GCH, 128), jnp.int32),
        pltpu.SemaphoreType.DMA,
        pltpu.SemaphoreType.DMA,
        pltpu.SemaphoreType.DMA,
        pltpu.SemaphoreType.DMA,
    ],
)
def _sc_sample_gather(video_hbm, audio_hbm, text_hbm, tg_hbm,
                      p0_hbm, p1_hbm, idx0_hbm, idx1_hbm, up_hbm, gf_hbm,
                      out_v, out_a, out_t, out_tg,
                      ptab, itab, ubuf, gbuf, ibuf,
                      vb0, vb1, ab0, ab1, tb0, tb1, gb0, gb1,
                      sin0, sin1, sout0, sout1):
    wid = lax.axis_index("s") * _NC + lax.axis_index("c")
    base = wid * _BPW

    pltpu.sync_copy(p0_hbm, ptab.at[pl.ds(0, _BATCH)])
    pltpu.sync_copy(p1_hbm, ptab.at[pl.ds(_BATCH, _BATCH)])
    pltpu.sync_copy(idx0_hbm, itab.at[pl.ds(0, _BATCH)])
    pltpu.sync_copy(idx1_hbm, itab.at[pl.ds(_BATCH, _BATCH)])
    pltpu.sync_copy(up_hbm.at[pl.ds(base, _BPW)], ubuf)
    pltpu.sync_copy(gf_hbm.at[pl.ds(base, _BPW)], gbuf)

    t0 = ptab[pl.ds(_BATCH - 16, 16)][15]
    t1 = ptab[pl.ds(2 * _BATCH - 16, 16)][15]
    iota = lax.broadcasted_iota(jnp.int32, (16,), 0)

    def search_body(c, carry):
        co = pl.multiple_of(c * _SCH, 16)
        upv = ubuf[pl.ds(co, 16)]
        gv = gbuf[pl.ds(co, 16)]
        rv = jnp.where(gv == 1, t1, t0) * (jnp.float32(1.0) - upv)
        tb = [gv[l] * _BATCH for l in range(16)]
        rs = [rv[l] for l in range(16)]
        pos = [jnp.int32(0)] * 16
        for k in _SEARCH_STEPS:
            for l in range(16):
                probe = ptab[pl.ds(tb[l] + pos[l] + (k - 1), 16)][0]
                pos[l] = jnp.where(probe < rs[l], pos[l] + k, pos[l])
        acc = jnp.zeros((16,), jnp.int32)
        for l in range(16):
            member = itab[pl.ds(tb[l] + pos[l], 16)][0]
            acc = acc + jnp.where(iota == l, member, 0)
        ibuf[pl.ds(co, 16)] = acc
        return carry

    lax.fori_loop(0, _BPW // _SCH, search_body, jnp.int32(0))

    def start_in(c, vb, ab, tb_, gb, sem):
        isl = ibuf.at[pl.ds(c * _GCH, _GCH)]
        h = [pltpu.async_copy(video_hbm.at[isl], vb, sem),
             pltpu.async_copy(audio_hbm.at[isl], ab, sem),
             pltpu.async_copy(text_hbm.at[isl], tb_, sem),
             pltpu.async_copy(tg_hbm.at[isl], gb, sem)]
        return h

    def start_out(c, vb, ab, tb_, gb, sem):
        row = base + c * _GCH
        pltpu.async_copy(vb, out_v.at[pl.ds(row, _GCH)], sem)
        pltpu.async_copy(ab, out_a.at[pl.ds(row, _GCH)], sem)
        pltpu.async_copy(tb_, out_t.at[pl.ds(row, _GCH)], sem)
        pltpu.async_copy(gb, out_tg.at[pl.ds(row, _GCH)], sem)

    def wait_out(vb, ab, tb_, gb, sem):
        pltpu.make_async_copy(vb, out_v.at[pl.ds(0, _GCH)], sem).wait()
        pltpu.make_async_copy(ab, out_a.at[pl.ds(0, _GCH)], sem).wait()
        pltpu.make_async_copy(tb_, out_t.at[pl.ds(0, _GCH)], sem).wait()
        pltpu.make_async_copy(gb, out_tg.at[pl.ds(0, _GCH)], sem).wait()

    def ring_body(i, carry):
        a = i * 2

        @pl.when(i > 0)
        def _():
            wait_out(vb0, ab0, tb0, gb0, sout0)
            wait_out(vb1, ab1, tb1, gb1, sout1)

        h0 = start_in(a, vb0, ab0, tb0, gb0, sin0)
        h1 = start_in(a + 1, vb1, ab1, tb1, gb1, sin1)
        for h in h0:
            h.wait()
        start_out(a, vb0, ab0, tb0, gb0, sout0)
        for h in h1:
            h.wait()
        start_out(a + 1, vb1, ab1, tb1, gb1, sout1)
        return carry

    lax.fori_loop(0, _BPW // (2 * _GCH), ring_body, jnp.int32(0))
    wait_out(vb0, ab0, tb0, gb0, sout0)
    wait_out(vb1, ab1, tb1, gb1, sout1)


def kernel(batch_video, batch_audio, batch_text, batch_target, batch_group,
           batch_group_others):
    positions = jnp.arange(_BATCH)
    n_cs = []
    per_weights = []
    for i in range(_NUM_GROUP):
        mask = batch_group == i
        n_c = jnp.sum(mask)
        n_cs.append(n_c)
        n_c_f = n_c.astype(jnp.float32)
        weights_list = []
        for j in range(4):
            n = jnp.sum(mask & (batch_group_others == j)).astype(jnp.float32)
            weights_list.append((n / n_c_f) ** _TAU)
        tot = sum(weights_list)
        weights_list = [w / tot for w in weights_list]
        per_weights.append(weights_list)
    go = batch_group_others
    wl0, wl1 = per_weights
    wfull = jnp.where(
        batch_group == 0,
        jnp.where(go == 0, wl0[0], jnp.where(go == 1, wl0[1],
                  jnp.where(go == 2, wl0[2], wl0[3]))),
        jnp.where(go == 0, wl1[0], jnp.where(go == 1, wl1[1],
                  jnp.where(go == 2, wl1[2], wl1[3]))))
    idx0, idx1, w0c, w1c = _sc_compact(batch_group, wfull)
    idx_gs = [idx0, idx1]
    p_cumls = []
    for i, w_arr in enumerate((w0c, w1c)):
        w_arr = jnp.where(positions < n_cs[i], w_arr, 0.0)
        probs = w_arr / w_arr.sum()
        p_cumls.append(jnp.cumsum(probs))

    tg_packed = jnp.pad(
        jnp.stack([batch_target, batch_group], axis=-1), ((0, 0), (0, 126)))
    video, audio, text, tg_out = _sc_sample_gather(
        batch_video, batch_audio, batch_text, tg_packed,
        p_cumls[0], p_cumls[1], idx_gs[0], idx_gs[1],
        jnp.asarray(_UP), jnp.asarray(_GF))
    return (video, audio, text, tg_out[:, 0], tg_out[:, 1])
